# Initial kernel scaffold; baseline (speedup 1.0000x reference)
import jax
import jax.numpy as jnp
from jax.experimental import pallas as pl


def kernel(metabolite_features, hyperedge_index, W_g1, b_g1, W_g2, b_g2, W_t, b_t):
    raise NotImplementedError("write your pallas kernel here")


# speedup vs baseline: 25.3500x; 25.3500x over previous
import functools

import jax
import jax.numpy as jnp
from jax import lax
from jax.experimental import pallas as pl
from jax.experimental.pallas import tpu as pltpu
from jax.experimental.pallas import tpu_sc as plsc

_H = 128
_N = 10000
_E = 320000
_SUB = 80
_NC = 2
_NS = 16
_NW = _NC * _NS
_EPT = _E // _NW
_JPT = _EPT // _SUB
_L = 16
_RPS = 624
_TAIL = _N - _NS * _RPS


def _prologue_body(x_ref, wg1_ref, bg1_ref, wg2_ref, bg2_ref, wt_ref,
                   bt_ref, r_ref, p_ref):
    x = x_ref[...]
    h = jnp.maximum(
        jnp.dot(x, wg1_ref[...], preferred_element_type=jnp.float32)
        + bg1_ref[...], 0.0)
    g = (jnp.dot(h, wg2_ref[...], preferred_element_type=jnp.float32)
         + bg2_ref[...])
    p = jnp.exp(g - jnp.max(g))
    t = jnp.maximum(
        jnp.dot(x, wt_ref[...], preferred_element_type=jnp.float32)
        + bt_ref[...], 0.0)
    r_ref[...] = p * t
    p_ref[...] = p


def _edge_body(r_hbm, p_hbm, src_hbm, dst_hbm, z2_hbm, z1_hbm,
               out_hbm, sout_hbm,
               src_v, dst_v, rows_v, p_v, s_tile, accum, gsem):
    c = lax.axis_index("c")
    s = lax.axis_index("s")
    wid = c * _NS + s
    pltpu.sync_copy(z2_hbm.at[pl.ds(s * _RPS, _RPS)],
                    accum.at[pl.ds(s * _RPS, _RPS)])

    @pl.when(s == 0)
    def _():
        pltpu.sync_copy(z2_hbm.at[pl.ds(_NS * _RPS, _TAIL)],
                        accum.at[pl.ds(_NS * _RPS, _TAIL)])

    pltpu.sync_copy(src_hbm.at[wid], src_v)
    pltpu.sync_copy(dst_hbm.at[wid], dst_v)
    pltpu.sync_copy(p_hbm, p_v)
    pltpu.sync_copy(z1_hbm, s_tile)
    plsc.subcore_barrier()

    def body(j, carry):
        pltpu.async_copy(r_hbm.at[src_v.at[j]], rows_v, gsem).wait()
        pltpu.sync_copy(rows_v, accum.at[dst_v.at[j]], add=True)
        srow = src_v.at[j]
        drow = dst_v.at[j]
        for k in range(_SUB // _L):
            sidx = srow[pl.ds(k * _L, _L)]
            didx = drow[pl.ds(k * _L, _L)]
            pv = plsc.load_gather(p_v, [sidx])
            plsc.addupdate_scatter(s_tile, [didx], pv)
        return carry

    lax.fori_loop(0, _JPT, body, 0)
    pltpu.sync_copy(s_tile, sout_hbm.at[wid])
    plsc.subcore_barrier()
    pltpu.sync_copy(accum.at[pl.ds(s * _RPS, _RPS)],
                    out_hbm.at[c].at[pl.ds(s * _RPS, _RPS)])

    @pl.when(s == 0)
    def _():
        pltpu.sync_copy(accum.at[pl.ds(_NS * _RPS, _TAIL)],
                        out_hbm.at[c].at[pl.ds(_NS * _RPS, _TAIL)])


def _epilogue_body(u_ref, sp_ref, z_ref):
    u = u_ref[0] + u_ref[1]
    den = jnp.sum(sp_ref[...], axis=0)
    z_ref[...] = u / (den[:, None] + 1e-16)


def kernel(metabolite_features, hyperedge_index, W_g1, b_g1, W_g2, b_g2,
           W_t, b_t):
    x = metabolite_features
    src = hyperedge_index[0].reshape(_NW, _JPT, _SUB)
    dst = hyperedge_index[1].reshape(_NW, _JPT, _SUB)

    r, p = pl.pallas_call(
        _prologue_body,
        out_shape=(jax.ShapeDtypeStruct((_N, _H), jnp.float32),
                   jax.ShapeDtypeStruct((_N, 1), jnp.float32)),
    )(x, W_g1, b_g1.reshape(1, -1), W_g2, b_g2.reshape(1, -1),
      W_t, b_t.reshape(1, -1))

    mesh = plsc.VectorSubcoreMesh(core_axis_name="c", subcore_axis_name="s")
    edge_kernel = functools.partial(
        pl.kernel,
        mesh=mesh,
        out_type=(jax.ShapeDtypeStruct((_NC, _N, _H), jnp.float32),
                  jax.ShapeDtypeStruct((_NW, _N), jnp.float32)),
        scratch_types=[
            pltpu.VMEM((_JPT, _SUB), jnp.int32),
            pltpu.VMEM((_JPT, _SUB), jnp.int32),
            pltpu.VMEM((_SUB, _H), jnp.float32),
            pltpu.VMEM((_N,), jnp.float32),
            pltpu.VMEM((_N,), jnp.float32),
            pltpu.VMEM_SHARED((_N, _H), jnp.float32),
            pltpu.SemaphoreType.DMA,
        ],
        compiler_params=pltpu.CompilerParams(
            needs_layout_passes=False, use_tc_tiling_on_sc=False),
    )(_edge_body)
    u, sp = edge_kernel(r, p.reshape(_N), src, dst,
                        jnp.zeros((_N, _H), jnp.float32),
                        jnp.zeros((_N,), jnp.float32))

    z = pl.pallas_call(
        _epilogue_body,
        out_shape=jax.ShapeDtypeStruct((_N, _H), jnp.float32),
    )(u, sp)
    return z


# speedup vs baseline: 26.8413x; 1.0588x over previous
import functools

import jax
import jax.numpy as jnp
from jax import lax
from jax.experimental import pallas as pl
from jax.experimental.pallas import tpu as pltpu
from jax.experimental.pallas import tpu_sc as plsc

_H = 128
_N = 10000
_E = 320000
_W = 144
_SUB = 50
_NC = 2
_NS = 16
_NW = _NC * _NS
_EPT = _E // _NW
_JPT = _EPT // _SUB
_NPAIR = _JPT // 2
_RPS = 624
_TAIL = _N - _NS * _RPS


def _prologue_body(x_ref, wg1_ref, bg1_ref, wg2_ref, bg2_ref, wt_ref,
                   bt_ref, r_ref):
    x = x_ref[...]
    h = jnp.maximum(
        jnp.dot(x, wg1_ref[...], preferred_element_type=jnp.float32)
        + bg1_ref[...], 0.0)
    g = (jnp.dot(h, wg2_ref[...], preferred_element_type=jnp.float32)
         + bg2_ref[...])
    p = jnp.exp(g - jnp.max(g))
    t = jnp.maximum(
        jnp.dot(x, wt_ref[...], preferred_element_type=jnp.float32)
        + bt_ref[...], 0.0)
    r_ref[:, :_H] = p * t
    r_ref[:, _H:_H + 1] = p
    r_ref[:, _H + 1:] = jnp.zeros((x.shape[0], _W - _H - 1), jnp.float32)


def _edge_body(r_hbm, src_hbm, dst_hbm, z2_hbm, out_hbm,
               src_v, dst_v, rows0, rows1, accum,
               gsem0, gsem1, ssem0, ssem1):
    c = lax.axis_index("c")
    s = lax.axis_index("s")
    wid = c * _NS + s
    if True:
        pltpu.sync_copy(z2_hbm.at[pl.ds(s * _RPS, _RPS)],
                        accum.at[pl.ds(s * _RPS, _RPS)])

        @pl.when(s == 0)
        def _():
            pltpu.sync_copy(z2_hbm.at[pl.ds(_NS * _RPS, _TAIL)],
                            accum.at[pl.ds(_NS * _RPS, _TAIL)])

        pltpu.sync_copy(src_hbm.at[wid], src_v)
        pltpu.sync_copy(dst_hbm.at[wid], dst_v)
        plsc.subcore_barrier()

        pltpu.async_copy(r_hbm.at[src_v.at[0]], rows0, gsem0)
        pltpu.async_copy(r_hbm.at[src_v.at[1]], rows1, gsem1)

        def step(j, rows, gsem, ssem):
            pltpu.make_async_copy(r_hbm.at[src_v.at[j]], rows, gsem).wait()
            pltpu.async_copy(rows, accum.at[dst_v.at[j]], ssem,
                             add=True).wait()

            @pl.when(j + 2 < _JPT)
            def _():
                pltpu.async_copy(r_hbm.at[src_v.at[j + 2]], rows, gsem)

        def body(k, carry):
            step(2 * k, rows0, gsem0, ssem0)
            step(2 * k + 1, rows1, gsem1, ssem1)
            return carry

        lax.fori_loop(0, _NPAIR, body, 0)
        plsc.subcore_barrier()
        pltpu.sync_copy(accum.at[pl.ds(s * _RPS, _RPS)],
                        out_hbm.at[c].at[pl.ds(s * _RPS, _RPS)])

        @pl.when(s == 0)
        def _():
            pltpu.sync_copy(accum.at[pl.ds(_NS * _RPS, _TAIL)],
                            out_hbm.at[c].at[pl.ds(_NS * _RPS, _TAIL)])


def _epilogue_body(u_ref, z_ref):
    u = u_ref[0] + u_ref[1]
    z_ref[...] = u[:, :_H] / (u[:, _H:_H + 1] + 1e-16)


def kernel(metabolite_features, hyperedge_index, W_g1, b_g1, W_g2, b_g2,
           W_t, b_t):
    x = metabolite_features
    src = hyperedge_index[0].reshape(_NW, _JPT, _SUB)
    dst = hyperedge_index[1].reshape(_NW, _JPT, _SUB)

    r = pl.pallas_call(
        _prologue_body,
        out_shape=jax.ShapeDtypeStruct((_N, _W), jnp.float32),
    )(x, W_g1, b_g1.reshape(1, -1), W_g2, b_g2.reshape(1, -1),
      W_t, b_t.reshape(1, -1))

    mesh = plsc.VectorSubcoreMesh(core_axis_name="c", subcore_axis_name="s")
    edge_kernel = functools.partial(
        pl.kernel,
        mesh=mesh,
        out_type=jax.ShapeDtypeStruct((_NC, _N, _W), jnp.float32),
        scratch_types=[
            pltpu.VMEM((_JPT, _SUB), jnp.int32),
            pltpu.VMEM((_JPT, _SUB), jnp.int32),
            pltpu.VMEM((_SUB, _W), jnp.float32),
            pltpu.VMEM((_SUB, _W), jnp.float32),
            pltpu.VMEM_SHARED((_N, _W), jnp.float32),
            pltpu.SemaphoreType.DMA,
            pltpu.SemaphoreType.DMA,
            pltpu.SemaphoreType.DMA,
            pltpu.SemaphoreType.DMA,
        ],
        compiler_params=pltpu.CompilerParams(
            needs_layout_passes=False, use_tc_tiling_on_sc=False),
    )(_edge_body)
    u = edge_kernel(r, src, dst, jnp.zeros((_N, _W), jnp.float32))

    z = pl.pallas_call(
        _epilogue_body,
        out_shape=jax.ShapeDtypeStruct((_N, _H), jnp.float32),
    )(u)
    return z
